# Initial kernel scaffold; baseline (speedup 1.0000x reference)
import jax
import jax.numpy as jnp
from jax.experimental import pallas as pl


def kernel(token_ids, w):
    raise NotImplementedError("write your pallas kernel here")


# speedup vs baseline: 1.7978x; 1.7978x over previous
import functools

import jax
import jax.numpy as jnp
from jax import lax
from jax.experimental import pallas as pl
from jax.experimental.pallas import tpu as pltpu
from jax.experimental.pallas import tpu_sc as plsc

NC, NS = 2, 16
NW = NC * NS
IW = 128
CPW_IR = 4


@functools.lru_cache(maxsize=None)
def _build(B, D):
    n_ir = B // IW
    ir_per_w = n_ir // NW
    n_chunks = ir_per_w // CPW_IR
    rows_per_chunk = CPW_IR * IW

    mesh = plsc.VectorSubcoreMesh(
        core_axis_name="c", subcore_axis_name="s",
        num_cores=NC, num_subcores=NS)

    @functools.partial(
        pl.kernel,
        mesh=mesh,
        compiler_params=pltpu.CompilerParams(use_tc_tiling_on_sc=False),
        out_type=jax.ShapeDtypeStruct((B, D), jnp.float32),
        scratch_types=[
            pltpu.VMEM((CPW_IR, IW), jnp.int32),
            pltpu.VMEM((rows_per_chunk, D), jnp.float32),
            pltpu.SemaphoreType.DMA,
        ],
    )
    def gather_kernel(idx_hbm, table_hbm, out_hbm, idx_v, rows_v, sem):
        wid = lax.axis_index("s") * NC + lax.axis_index("c")
        ir_base = wid * ir_per_w

        def body(c, carry):
            ir0 = ir_base + c * CPW_IR
            pltpu.sync_copy(idx_hbm.at[pl.ds(ir0, CPW_IR)], idx_v)
            cps = [
                pltpu.async_copy(
                    table_hbm.at[idx_v.at[j]],
                    rows_v.at[pl.ds(j * IW, IW)],
                    sem)
                for j in range(CPW_IR)
            ]
            for cp in cps:
                cp.wait()
            pltpu.sync_copy(rows_v, out_hbm.at[pl.ds(ir0 * IW, rows_per_chunk)])
            return carry

        lax.fori_loop(0, n_chunks, body, 0)

    return gather_kernel


def kernel(token_ids, w):
    B = token_ids.shape[0] * token_ids.shape[1]
    flat = token_ids.reshape(B // IW, IW).astype(jnp.int32)
    out = _build(B, w.shape[1])(flat, w)
    return out.reshape(*token_ids.shape, w.shape[1])


# speedup vs baseline: 1.8530x; 1.0307x over previous
import functools

import jax
import jax.numpy as jnp
from jax import lax
from jax.experimental import pallas as pl
from jax.experimental.pallas import tpu as pltpu
from jax.experimental.pallas import tpu_sc as plsc

NC, NS = 2, 16
NW = NC * NS
IW = 128
CPW_IR = 4
RPC = CPW_IR * IW


@functools.lru_cache(maxsize=None)
def _build(B, D):
    n_ir = B // IW
    ir_per_w = n_ir // NW
    n_chunks = ir_per_w // CPW_IR
    assert n_chunks % 2 == 0 and n_chunks >= 6

    mesh = plsc.VectorSubcoreMesh(
        core_axis_name="c", subcore_axis_name="s",
        num_cores=NC, num_subcores=NS)

    @functools.partial(
        pl.kernel,
        mesh=mesh,
        compiler_params=pltpu.CompilerParams(use_tc_tiling_on_sc=False),
        out_type=jax.ShapeDtypeStruct((B, D), jnp.float32),
        scratch_types=[
            pltpu.VMEM((2, CPW_IR, IW), jnp.int32),
            pltpu.VMEM((2, RPC, D), jnp.float32),
            pltpu.SemaphoreType.DMA((2,)),
            pltpu.SemaphoreType.DMA((2,)),
        ],
    )
    def gather_kernel(idx_hbm, table_hbm, out_hbm, idx_v, rows_v, gsem, osem):
        wid = lax.axis_index("s") * NC + lax.axis_index("c")
        ir_base = wid * ir_per_w

        def fire(c, b):
            ir0 = ir_base + c * CPW_IR
            pltpu.sync_copy(idx_hbm.at[pl.ds(ir0, CPW_IR)], idx_v.at[b])
            for j in range(CPW_IR):
                pltpu.async_copy(
                    table_hbm.at[idx_v.at[b, j]],
                    rows_v.at[b, pl.ds(j * IW, IW)],
                    gsem.at[b])

        def retire(c, b):
            ir0 = ir_base + c * CPW_IR
            row0 = ir0 * IW
            pltpu.make_async_copy(
                out_hbm.at[pl.ds(row0, RPC)], rows_v.at[b], gsem.at[b]).wait()
            pltpu.async_copy(
                rows_v.at[b], out_hbm.at[pl.ds(row0, RPC)], osem.at[b])

        def drain_out(b):
            pltpu.make_async_copy(
                out_hbm.at[pl.ds(ir_base * IW, RPC)], rows_v.at[b],
                osem.at[b]).wait()

        fire(0, 0)
        fire(1, 1)
        retire(0, 0)
        drain_out(0)
        fire(2, 0)
        retire(1, 1)

        def body(g, carry):
            c0 = 2 * g
            drain_out(1)
            fire(c0 + 1, 1)
            retire(c0, 0)
            drain_out(0)
            fire(c0 + 2, 0)
            retire(c0 + 1, 1)
            return carry

        lax.fori_loop(1, n_chunks // 2 - 1, body, 0)

        c0 = n_chunks - 2
        drain_out(1)
        fire(c0 + 1, 1)
        retire(c0, 0)
        retire(c0 + 1, 1)
        drain_out(0)
        drain_out(1)

    return gather_kernel


def kernel(token_ids, w):
    B = token_ids.shape[0] * token_ids.shape[1]
    flat = token_ids.reshape(B // IW, IW).astype(jnp.int32)
    out = _build(B, w.shape[1])(flat, w)
    return out.reshape(*token_ids.shape, w.shape[1])
